# Initial kernel scaffold; baseline (speedup 1.0000x reference)
import jax
import jax.numpy as jnp
from jax.experimental import pallas as pl


def kernel(x, index, Wg, bg, Wm, bm):
    raise NotImplementedError("write your pallas kernel here")


# speedup vs baseline: 2.1955x; 2.1955x over previous
import jax
import jax.numpy as jnp
from jax.experimental import pallas as pl
from jax.experimental.pallas import tpu as pltpu

S = 1024
T = 2048
SB = S // 128


def _col_ids(b):
    ii = jax.lax.broadcasted_iota(jnp.int32, (T, 128), 1)
    return ii.astype(jnp.float32) + jnp.float32(128 * b)


def _ka(x_ref, idx_ref, wg_ref, bg_ref, gate_ref, gmax_ref):
    t = pl.program_id(0)

    @pl.when(t == 0)
    def _():
        gmax_ref[...] = jnp.full((SB, 128), -jnp.inf, jnp.float32)

    gate = jnp.dot(x_ref[...], wg_ref[...], preferred_element_type=jnp.float32)
    gate = gate + bg_ref[0, 0]
    gate_ref[...] = gate
    idxc = idx_ref[...]
    gateb = jnp.broadcast_to(gate, (T, 128))
    for b in range(SB):
        oh = idxc == _col_ids(b)
        m = jnp.max(jnp.where(oh, gateb, -jnp.inf), axis=0, keepdims=True)
        gmax_ref[b : b + 1, :] = jnp.maximum(gmax_ref[b : b + 1, :], m)


def _kb(gate_ref, idx_ref, gmax_ref, gsum_ref):
    t = pl.program_id(0)

    @pl.when(t == 0)
    def _():
        gsum_ref[...] = jnp.zeros((SB, 128), jnp.float32)

    gm_safe = jnp.maximum(gmax_ref[...], -1e30)
    idxc = idx_ref[...]
    gate = gate_ref[...]
    gmr = jnp.zeros((T, 1), jnp.float32)
    for b in range(SB):
        ohf = (idxc == _col_ids(b)).astype(jnp.float32)
        gmr = gmr + jnp.sum(ohf * gm_safe[b : b + 1, :], axis=1, keepdims=True)
    eb = jnp.broadcast_to(jnp.exp(gate - gmr), (T, 128))
    for b in range(SB):
        oh = idxc == _col_ids(b)
        s = jnp.sum(jnp.where(oh, eb, 0.0), axis=0, keepdims=True)
        gsum_ref[b : b + 1, :] = gsum_ref[b : b + 1, :] + s


def _kc(x_ref, gate_ref, idx_ref, gmax_ref, gsum_ref, wm_ref, bm_ref,
        out_ref, acc_ref, sw_ref):
    t = pl.program_id(0)
    nt = pl.num_programs(0)

    @pl.when(t == 0)
    def _():
        acc_ref[...] = jnp.zeros((S, 128), jnp.float32)
        sw_ref[...] = jnp.zeros((S, 1), jnp.float32)

    gm_safe = jnp.maximum(gmax_ref[...], -1e30)
    gs = gsum_ref[...]
    idxc = idx_ref[...]
    gate = gate_ref[...]
    gmr = jnp.zeros((T, 1), jnp.float32)
    gsr = jnp.zeros((T, 1), jnp.float32)
    for b in range(SB):
        ohf = (idxc == _col_ids(b)).astype(jnp.float32)
        gmr = gmr + jnp.sum(ohf * gm_safe[b : b + 1, :], axis=1, keepdims=True)
        gsr = gsr + jnp.sum(ohf * gs[b : b + 1, :], axis=1, keepdims=True)
    w = jnp.exp(gate - gmr) / (gsr + 1e-10)
    wb = jnp.broadcast_to(w, (T, 128))
    ones = jnp.ones((T, 1), jnp.float32)
    xt = x_ref[...]
    cdims = (((0,), (0,)), ((), ()))
    for b in range(SB):
        ohw = jnp.where(idxc == _col_ids(b), wb, 0.0)
        acc_ref[pl.ds(128 * b, 128), :] += jax.lax.dot_general(
            ohw, xt, cdims, preferred_element_type=jnp.float32)
        sw_ref[pl.ds(128 * b, 128), :] += jax.lax.dot_general(
            ohw, ones, cdims, preferred_element_type=jnp.float32)

    @pl.when(t == nt - 1)
    def _():
        out_ref[...] = (
            jnp.dot(acc_ref[...], wm_ref[...], preferred_element_type=jnp.float32)
            + sw_ref[...] * bm_ref[...]
        )


def kernel(x, index, Wg, bg, Wm, bm):
    N, D = x.shape
    nt = (N + T - 1) // T
    n_pad = nt * T
    xp = jnp.pad(x, ((0, n_pad - N), (0, 0)))
    idxf = jnp.pad(index, (0, n_pad - N), constant_values=-1)
    idxf = idxf.astype(jnp.float32).reshape(n_pad, 1)
    bg2 = bg.reshape(1, 1)
    bm2 = bm.reshape(1, D)

    gate, gmax = pl.pallas_call(
        _ka,
        grid=(nt,),
        in_specs=[
            pl.BlockSpec((T, D), lambda i: (i, 0)),
            pl.BlockSpec((T, 1), lambda i: (i, 0)),
            pl.BlockSpec((D, 1), lambda i: (0, 0)),
            pl.BlockSpec((1, 1), lambda i: (0, 0)),
        ],
        out_specs=[
            pl.BlockSpec((T, 1), lambda i: (i, 0)),
            pl.BlockSpec((SB, 128), lambda i: (0, 0)),
        ],
        out_shape=[
            jax.ShapeDtypeStruct((n_pad, 1), jnp.float32),
            jax.ShapeDtypeStruct((SB, 128), jnp.float32),
        ],
    )(xp, idxf, Wg, bg2)

    gsum = pl.pallas_call(
        _kb,
        grid=(nt,),
        in_specs=[
            pl.BlockSpec((T, 1), lambda i: (i, 0)),
            pl.BlockSpec((T, 1), lambda i: (i, 0)),
            pl.BlockSpec((SB, 128), lambda i: (0, 0)),
        ],
        out_specs=pl.BlockSpec((SB, 128), lambda i: (0, 0)),
        out_shape=jax.ShapeDtypeStruct((SB, 128), jnp.float32),
    )(gate, idxf, gmax)

    out = pl.pallas_call(
        _kc,
        grid=(nt,),
        in_specs=[
            pl.BlockSpec((T, D), lambda i: (i, 0)),
            pl.BlockSpec((T, 1), lambda i: (i, 0)),
            pl.BlockSpec((T, 1), lambda i: (i, 0)),
            pl.BlockSpec((SB, 128), lambda i: (0, 0)),
            pl.BlockSpec((SB, 128), lambda i: (0, 0)),
            pl.BlockSpec((D, D), lambda i: (0, 0)),
            pl.BlockSpec((1, D), lambda i: (0, 0)),
        ],
        out_specs=pl.BlockSpec((S, D), lambda i: (0, 0)),
        out_shape=jax.ShapeDtypeStruct((S, D), jnp.float32),
        scratch_shapes=[
            pltpu.VMEM((S, 128), jnp.float32),
            pltpu.VMEM((S, 1), jnp.float32),
        ],
    )(xp, gate, idxf, gmax, gsum, Wm, bm2)

    return out


# speedup vs baseline: 3.4719x; 1.5814x over previous
import functools

import jax
import jax.numpy as jnp
from jax import lax
from jax.experimental import pallas as pl
from jax.experimental.pallas import tpu as pltpu
from jax.experimental.pallas import tpu_sc as plsc

S = 1024
T = 2048
SB = S // 128
NW = 32
CH = 128
NEG = -1e30


def _col_ids(b):
    ii = jax.lax.broadcasted_iota(jnp.int32, (T, 128), 1)
    return ii.astype(jnp.float32) + jnp.float32(128 * b)


def _ka(x_ref, idx_ref, wg_ref, bg_ref, gate_ref, gmax_ref):
    t = pl.program_id(0)

    @pl.when(t == 0)
    def _():
        gmax_ref[...] = jnp.full((SB, 128), -jnp.inf, jnp.float32)

    gate = jnp.dot(x_ref[...], wg_ref[...], preferred_element_type=jnp.float32)
    gate = gate + bg_ref[0, 0]
    idxc = idx_ref[...]
    gate_ref[...] = jnp.where(idxc < 0, NEG, gate)
    gateb = jnp.broadcast_to(gate, (T, 128))
    for b in range(SB):
        oh = idxc == _col_ids(b)
        m = jnp.max(jnp.where(oh, gateb, -jnp.inf), axis=0, keepdims=True)
        gmax_ref[b : b + 1, :] = jnp.maximum(gmax_ref[b : b + 1, :], m)


def _kb(gate_ref, idx_ref, gmax_ref, gsum_ref):
    t = pl.program_id(0)

    @pl.when(t == 0)
    def _():
        gsum_ref[...] = jnp.zeros((SB, 128), jnp.float32)

    gm_safe = jnp.maximum(gmax_ref[...], NEG)
    idxc = idx_ref[...]
    gate = gate_ref[...]
    gmr = jnp.zeros((T, 1), jnp.float32)
    for b in range(SB):
        ohf = (idxc == _col_ids(b)).astype(jnp.float32)
        gmr = gmr + jnp.sum(ohf * gm_safe[b : b + 1, :], axis=1, keepdims=True)
    eb = jnp.broadcast_to(jnp.exp(gate - gmr), (T, 128))
    for b in range(SB):
        oh = idxc == _col_ids(b)
        s = jnp.sum(jnp.where(oh, eb, 0.0), axis=0, keepdims=True)
        gsum_ref[b : b + 1, :] = gsum_ref[b : b + 1, :] + s


def _ksc(x_hbm, idx_hbm, gate_hbm, gmax_hbm, gsum_hbm, zeros_hbm, out_hbm,
         xbuf, ibuf, gbuf, wbuf, gmax_v, gsum_v, acc_sh):
    c = lax.axis_index("c")
    sid = lax.axis_index("s")
    wid = c * 16 + sid
    nch = idx_hbm.shape[1]
    rows_per_w = nch * CH

    @pl.when(sid == 0)
    def _():
        pltpu.sync_copy(zeros_hbm, acc_sh)

    pltpu.sync_copy(idx_hbm.at[wid], ibuf)
    pltpu.sync_copy(gate_hbm.at[wid], gbuf)
    pltpu.sync_copy(gmax_hbm, gmax_v)
    pltpu.sync_copy(gsum_hbm, gsum_v)
    plsc.subcore_barrier()

    base = wid * rows_per_w

    def chunk_body(j, carry):
        pltpu.sync_copy(x_hbm.at[pl.ds(base + j * CH, CH), :], xbuf)
        for g in range(CH // 16):
            iv = ibuf[j, pl.ds(g * 16, 16)]
            gv = gbuf[j, pl.ds(g * 16, 16)]
            gm = plsc.load_gather(gmax_v, [iv])
            gs = plsc.load_gather(gsum_v, [iv])
            wbuf[pl.ds(g * 16, 16)] = jnp.exp(gv - gm) / (gs + 1e-10)

        for g in range(CH // 16):
            w16 = wbuf[pl.ds(g * 16, 16)]
            for l in range(16):
                r = g * 16 + l
                wv = w16[l]
                for k in range(8):
                    xbuf[r, pl.ds(k * 16, 16)] = xbuf[r, pl.ds(k * 16, 16)] * wv
        pltpu.sync_copy(xbuf, acc_sh.at[ibuf.at[j]], add=True)
        return carry

    lax.fori_loop(0, nch, chunk_body, 0)
    plsc.subcore_barrier()

    @pl.when(sid == 0)
    def _():
        pltpu.sync_copy(acc_sh, out_hbm.at[c])


def _kd(p_ref, gsum_ref, wm_ref, bm_ref, out_ref):
    psum = p_ref[0] + p_ref[1]
    out = jnp.dot(psum, wm_ref[...], preferred_element_type=jnp.float32)
    sw = gsum_ref[...]
    sw = sw / (sw + 1e-10)
    one = jnp.ones((1, 1), jnp.float32)
    cdims = (((0,), (0,)), ((), ()))
    cols = [
        jax.lax.dot_general(sw[b : b + 1, :], one, cdims,
                            preferred_element_type=jnp.float32)
        for b in range(SB)
    ]
    swcol = jnp.concatenate(cols, axis=0)
    out_ref[...] = out + swcol * bm_ref[...]


def kernel(x, index, Wg, bg, Wm, bm):
    N, D = x.shape
    n_pad = NW * 3200
    nt = n_pad // T
    nch = n_pad // (NW * CH)
    xp = jnp.pad(x, ((0, n_pad - N), (0, 0)))
    idxp = jnp.pad(index, (0, n_pad - N), constant_values=-1)
    idxf = idxp.astype(jnp.float32).reshape(n_pad, 1)
    idx3 = jnp.maximum(idxp, 0).reshape(NW, nch, CH)
    bg2 = bg.reshape(1, 1)
    bm2 = bm.reshape(1, D)

    gate, gmax = pl.pallas_call(
        _ka,
        grid=(nt,),
        in_specs=[
            pl.BlockSpec((T, D), lambda i: (i, 0)),
            pl.BlockSpec((T, 1), lambda i: (i, 0)),
            pl.BlockSpec((D, 1), lambda i: (0, 0)),
            pl.BlockSpec((1, 1), lambda i: (0, 0)),
        ],
        out_specs=[
            pl.BlockSpec((T, 1), lambda i: (i, 0)),
            pl.BlockSpec((SB, 128), lambda i: (0, 0)),
        ],
        out_shape=[
            jax.ShapeDtypeStruct((n_pad, 1), jnp.float32),
            jax.ShapeDtypeStruct((SB, 128), jnp.float32),
        ],
    )(xp, idxf, Wg, bg2)

    gsum = pl.pallas_call(
        _kb,
        grid=(nt,),
        in_specs=[
            pl.BlockSpec((T, 1), lambda i: (i, 0)),
            pl.BlockSpec((T, 1), lambda i: (i, 0)),
            pl.BlockSpec((SB, 128), lambda i: (0, 0)),
        ],
        out_specs=pl.BlockSpec((SB, 128), lambda i: (0, 0)),
        out_shape=jax.ShapeDtypeStruct((SB, 128), jnp.float32),
    )(gate, idxf, gmax)

    gmax_flat = jnp.maximum(gmax.reshape(S), NEG)
    gsum_flat = gsum.reshape(S)
    gate3 = gate.reshape(NW, nch, CH)
    zeros = jnp.zeros((S, D), jnp.float32)

    sc = pl.kernel(
        _ksc,
        mesh=plsc.VectorSubcoreMesh(core_axis_name="c", subcore_axis_name="s"),
        out_type=jax.ShapeDtypeStruct((2, S, D), jnp.float32),
        scratch_types=[
            pltpu.VMEM((CH, D), jnp.float32),
            pltpu.VMEM((nch, CH), jnp.int32),
            pltpu.VMEM((nch, CH), jnp.float32),
            pltpu.VMEM((CH,), jnp.float32),
            pltpu.VMEM((S,), jnp.float32),
            pltpu.VMEM((S,), jnp.float32),
            pltpu.VMEM_SHARED((S, D), jnp.float32),
        ],
        compiler_params=pltpu.CompilerParams(needs_layout_passes=False),
    )
    partials = sc(xp, idx3, gate3, gmax_flat, gsum_flat, zeros)

    out = pl.pallas_call(
        _kd,
        grid=(1,),
        in_specs=[
            pl.BlockSpec((2, S, D), lambda i: (0, 0, 0)),
            pl.BlockSpec((SB, 128), lambda i: (0, 0)),
            pl.BlockSpec((D, D), lambda i: (0, 0)),
            pl.BlockSpec((1, D), lambda i: (0, 0)),
        ],
        out_specs=pl.BlockSpec((S, D), lambda i: (0, 0)),
        out_shape=jax.ShapeDtypeStruct((S, D), jnp.float32),
    )(partials, gsum, Wm, bm2)

    return out


# speedup vs baseline: 3.4956x; 1.0068x over previous
import functools

import jax
import jax.numpy as jnp
from jax import lax
from jax.experimental import pallas as pl
from jax.experimental.pallas import tpu as pltpu
from jax.experimental.pallas import tpu_sc as plsc

S = 1024
T = 2048
SB = S // 128
NW = 32
CH = 128
NEG = -1e30


def _col_ids(b):
    ii = jax.lax.broadcasted_iota(jnp.int32, (T, 128), 1)
    return ii.astype(jnp.float32) + jnp.float32(128 * b)


def _ka(x_ref, idx_ref, wg_ref, bg_ref, gate_ref, gmax_ref):
    t = pl.program_id(0)

    @pl.when(t == 0)
    def _():
        gmax_ref[...] = jnp.full((SB, 128), -jnp.inf, jnp.float32)

    gate = jnp.dot(x_ref[...], wg_ref[...], preferred_element_type=jnp.float32)
    gate = gate + bg_ref[0, 0]
    idxc = idx_ref[...]
    gate_ref[...] = jnp.where(idxc < 0, NEG, gate)
    gateb = jnp.broadcast_to(gate, (T, 128))
    lo = idx_ref[0, 0]
    hi = jnp.max(idxc)
    for b in range(SB):
        @pl.when(jnp.logical_and(hi >= 128.0 * b, lo < 128.0 * (b + 1)))
        def _(b=b):
            oh = idxc == _col_ids(b)
            m = jnp.max(jnp.where(oh, gateb, -jnp.inf), axis=0, keepdims=True)
            gmax_ref[b : b + 1, :] = jnp.maximum(gmax_ref[b : b + 1, :], m)


def _kb(gate_ref, idx_ref, gmax_ref, gsum_ref):
    t = pl.program_id(0)

    @pl.when(t == 0)
    def _():
        gsum_ref[...] = jnp.zeros((SB, 128), jnp.float32)

    gm_safe = jnp.maximum(gmax_ref[...], NEG)
    idxc = idx_ref[...]
    gate = gate_ref[...]
    gmr = jnp.zeros((T, 1), jnp.float32)
    for b in range(SB):
        ohf = (idxc == _col_ids(b)).astype(jnp.float32)
        gmr = gmr + jnp.sum(ohf * gm_safe[b : b + 1, :], axis=1, keepdims=True)
    eb = jnp.broadcast_to(jnp.exp(gate - gmr), (T, 128))
    for b in range(SB):
        oh = idxc == _col_ids(b)
        s = jnp.sum(jnp.where(oh, eb, 0.0), axis=0, keepdims=True)
        gsum_ref[b : b + 1, :] = gsum_ref[b : b + 1, :] + s


def _ksc(x_hbm, idx_hbm, gate_hbm, gmax_hbm, gsum_hbm, zeros_hbm, out_hbm,
         xbuf, ibuf, gbuf, wbuf, gmax_v, gsum_v, acc_sh):
    c = lax.axis_index("c")
    sid = lax.axis_index("s")
    wid = c * 16 + sid
    nch = idx_hbm.shape[1]
    rows_per_w = nch * CH

    @pl.when(sid == 0)
    def _():
        pltpu.sync_copy(zeros_hbm, acc_sh)

    pltpu.sync_copy(idx_hbm.at[wid], ibuf)
    pltpu.sync_copy(gate_hbm.at[wid], gbuf)
    pltpu.sync_copy(gmax_hbm, gmax_v)
    pltpu.sync_copy(gsum_hbm, gsum_v)
    plsc.subcore_barrier()

    base = wid * rows_per_w

    def chunk_body(j, carry):
        pltpu.sync_copy(x_hbm.at[pl.ds(base + j * CH, CH), :], xbuf)
        for g in range(CH // 16):
            iv = ibuf[j, pl.ds(g * 16, 16)]
            gv = gbuf[j, pl.ds(g * 16, 16)]
            gm = plsc.load_gather(gmax_v, [iv])
            gs = plsc.load_gather(gsum_v, [iv])
            wbuf[pl.ds(g * 16, 16)] = jnp.exp(gv - gm) / (gs + 1e-10)

        for g in range(CH // 16):
            w16 = wbuf[pl.ds(g * 16, 16)]
            for l in range(16):
                r = g * 16 + l
                wv = w16[l]
                for k in range(8):
                    xbuf[r, pl.ds(k * 16, 16)] = xbuf[r, pl.ds(k * 16, 16)] * wv
        pltpu.sync_copy(xbuf, acc_sh.at[ibuf.at[j]], add=True)
        return carry

    lax.fori_loop(0, nch, chunk_body, 0)
    plsc.subcore_barrier()

    @pl.when(sid == 0)
    def _():
        pltpu.sync_copy(acc_sh, out_hbm.at[c])


def _kd(p_ref, gsum_ref, wm_ref, bm_ref, out_ref):
    psum = p_ref[0] + p_ref[1]
    out = jnp.dot(psum, wm_ref[...], preferred_element_type=jnp.float32)
    sw = gsum_ref[...]
    sw = sw / (sw + 1e-10)
    one = jnp.ones((1, 1), jnp.float32)
    cdims = (((0,), (0,)), ((), ()))
    cols = [
        jax.lax.dot_general(sw[b : b + 1, :], one, cdims,
                            preferred_element_type=jnp.float32)
        for b in range(SB)
    ]
    swcol = jnp.concatenate(cols, axis=0)
    out_ref[...] = out + swcol * bm_ref[...]


def kernel(x, index, Wg, bg, Wm, bm):
    N, D = x.shape
    n_pad = NW * 3200
    nt = n_pad // T
    nch = n_pad // (NW * CH)
    xp = jnp.pad(x, ((0, n_pad - N), (0, 0)))
    idxp = jnp.pad(index, (0, n_pad - N), constant_values=-1)
    idxf = idxp.astype(jnp.float32).reshape(n_pad, 1)
    idx3 = jnp.maximum(idxp, 0).reshape(NW, nch, CH)
    bg2 = bg.reshape(1, 1)
    bm2 = bm.reshape(1, D)

    gate, gmax = pl.pallas_call(
        _ka,
        grid=(nt,),
        in_specs=[
            pl.BlockSpec((T, D), lambda i: (i, 0)),
            pl.BlockSpec((T, 1), lambda i: (i, 0)),
            pl.BlockSpec((D, 1), lambda i: (0, 0)),
            pl.BlockSpec((1, 1), lambda i: (0, 0)),
        ],
        out_specs=[
            pl.BlockSpec((T, 1), lambda i: (i, 0)),
            pl.BlockSpec((SB, 128), lambda i: (0, 0)),
        ],
        out_shape=[
            jax.ShapeDtypeStruct((n_pad, 1), jnp.float32),
            jax.ShapeDtypeStruct((SB, 128), jnp.float32),
        ],
    )(xp, idxf, Wg, bg2)

    gsum = pl.pallas_call(
        _kb,
        grid=(nt,),
        in_specs=[
            pl.BlockSpec((T, 1), lambda i: (i, 0)),
            pl.BlockSpec((T, 1), lambda i: (i, 0)),
            pl.BlockSpec((SB, 128), lambda i: (0, 0)),
        ],
        out_specs=pl.BlockSpec((SB, 128), lambda i: (0, 0)),
        out_shape=jax.ShapeDtypeStruct((SB, 128), jnp.float32),
    )(gate, idxf, gmax)

    gmax_flat = jnp.maximum(gmax.reshape(S), NEG)
    gsum_flat = gsum.reshape(S)
    gate3 = gate.reshape(NW, nch, CH)
    zeros = jnp.zeros((S, D), jnp.float32)

    sc = pl.kernel(
        _ksc,
        mesh=plsc.VectorSubcoreMesh(core_axis_name="c", subcore_axis_name="s"),
        out_type=jax.ShapeDtypeStruct((2, S, D), jnp.float32),
        scratch_types=[
            pltpu.VMEM((CH, D), jnp.float32),
            pltpu.VMEM((nch, CH), jnp.int32),
            pltpu.VMEM((nch, CH), jnp.float32),
            pltpu.VMEM((CH,), jnp.float32),
            pltpu.VMEM((S,), jnp.float32),
            pltpu.VMEM((S,), jnp.float32),
            pltpu.VMEM_SHARED((S, D), jnp.float32),
        ],
        compiler_params=pltpu.CompilerParams(needs_layout_passes=False),
    )
    partials = sc(xp, idx3, gate3, gmax_flat, gsum_flat, zeros)

    out = pl.pallas_call(
        _kd,
        grid=(1,),
        in_specs=[
            pl.BlockSpec((2, S, D), lambda i: (0, 0, 0)),
            pl.BlockSpec((SB, 128), lambda i: (0, 0)),
            pl.BlockSpec((D, D), lambda i: (0, 0)),
            pl.BlockSpec((1, D), lambda i: (0, 0)),
        ],
        out_specs=pl.BlockSpec((S, D), lambda i: (0, 0)),
        out_shape=jax.ShapeDtypeStruct((S, D), jnp.float32),
    )(partials, gsum, Wm, bm2)

    return out
